# baseline (speedup 1.0000x reference)
import functools

import jax
import jax.numpy as jnp
from jax import lax
from jax.experimental import pallas as pl
from jax.experimental.pallas import tpu as pltpu
from jax.experimental.pallas import tpu_sc as plsc

B, N1, N2 = 8, 1024, 8192
C1, C2 = 256, 128
XC = 3 + C2
OUTC = 3 + C1 + C2
TILE = 512
NT = N2 // TILE
ROWS = B * N2

NC, NS, L = 2, 16, 16
NW = NC * NS
ROWS_PER_W = ROWS // NW
CHUNK = 128
NCHUNK = ROWS_PER_W // CHUNK


def _argmin_body(xyz1t_ref, x1_ref, idx_ref):
    b = pl.program_id(0)
    xyz1t = xyz1t_ref[0]
    xyz2 = x1_ref[0][:, 0:3]
    cross = jnp.dot(xyz2, xyz1t, preferred_element_type=jnp.float32)
    x2sq = jnp.sum(xyz2 * xyz2, axis=1, keepdims=True)
    x1sq = jnp.sum(xyz1t * xyz1t, axis=0, keepdims=True)
    d = x2sq - 2.0 * cross + x1sq
    idx_ref[0, 0, :] = jnp.argmin(d, axis=1) + b * N1


def _sc_gather_body(f1_hbm, x1_hbm, idx_hbm, out_hbm, idx_v, g_v, x1_v, fat_v, sem):
    wid = lax.axis_index("s") * NC + lax.axis_index("c")
    base = wid * ROWS_PER_W

    lanes = lax.iota(jnp.int32, L)
    xcols = [lanes + k * L for k in range(9)]
    xocols = [jnp.where(c < 3, c, c + C1) for c in xcols]
    xmasks = [c < XC for c in xcols]
    gcols = [lanes + k * L for k in range(16)]
    gocols = [c + 3 for c in gcols]

    def chunk_body(k, _):
        off = base + k * CHUNK
        pltpu.sync_copy(idx_hbm.at[pl.ds(off, CHUNK)], idx_v)
        gat = pltpu.async_copy(f1_hbm.at[idx_v], g_v, sem)
        pltpu.sync_copy(x1_hbm.at[pl.ds(off, CHUNK)], x1_v)
        gat.wait()

        def row_body(i, _):
            row = jnp.broadcast_to(i, (L,)).astype(jnp.int32)
            for c, oc in zip(gcols, gocols):
                v = plsc.load_gather(g_v, [row, c])
                plsc.store_scatter(fat_v, [row, oc], v)
            for c, oc, m in zip(xcols, xocols, xmasks):
                v = plsc.load_gather(x1_v, [row, c], mask=m)
                plsc.store_scatter(fat_v, [row, oc], v, mask=m)
            return 0

        lax.fori_loop(0, CHUNK, row_body, 0)
        pltpu.sync_copy(fat_v, out_hbm.at[pl.ds(off, CHUNK)])
        return 0

    lax.fori_loop(0, NCHUNK, chunk_body, 0)


def kernel(x0, x1):
    xyz1t = jnp.transpose(x0[:, :, 0:3], (0, 2, 1))
    f1_flat = jnp.reshape(x0[:, :, 3:], (B * N1, C1))
    x1_flat = jnp.reshape(x1, (ROWS, XC))

    idxq = pl.pallas_call(
        _argmin_body,
        grid=(B, NT),
        in_specs=[
            pl.BlockSpec((1, 3, N1), lambda b, t: (b, 0, 0)),
            pl.BlockSpec((1, TILE, XC), lambda b, t: (b, t, 0)),
        ],
        out_specs=pl.BlockSpec((1, 1, TILE), lambda b, t: (b * NT + t, 0, 0)),
        out_shape=jax.ShapeDtypeStruct((B * NT, 1, TILE), jnp.int32),
    )(xyz1t, x1)
    idx_flat = jnp.reshape(idxq, (ROWS,))

    mesh = plsc.VectorSubcoreMesh(core_axis_name="c", subcore_axis_name="s")
    out_flat = pl.kernel(
        _sc_gather_body,
        out_type=jax.ShapeDtypeStruct((ROWS, OUTC), jnp.float32),
        mesh=mesh,
        compiler_params=pltpu.CompilerParams(
            use_tc_tiling_on_sc=False, needs_layout_passes=False),
        scratch_types=[
            pltpu.VMEM((CHUNK,), jnp.int32),
            pltpu.VMEM((CHUNK, C1), jnp.float32),
            pltpu.VMEM((CHUNK, XC), jnp.float32),
            pltpu.VMEM((CHUNK, OUTC), jnp.float32),
            pltpu.SemaphoreType.DMA,
        ],
    )(f1_flat, x1_flat, idx_flat)

    return (jnp.reshape(out_flat, (B, N2, OUTC)), x1[:, :, 0:3])


# speedup vs baseline: 2.1007x; 2.1007x over previous
import jax
import jax.numpy as jnp
from jax.experimental import pallas as pl

B, N1, N2 = 8, 1024, 8192
C1, C2 = 256, 128
OUTC = 3 + C1 + C2
TILE = 512


def _fused_body(xyz1t_ref, f1_ref, x1_ref, out_ref):
    xyz1t = xyz1t_ref[0]
    f1 = f1_ref[0]
    x1b = x1_ref[0]
    xyz2 = x1b[:, 0:3]
    f2 = x1b[:, 3:]

    cross = jnp.dot(xyz2, xyz1t, preferred_element_type=jnp.float32)
    x2sq = jnp.sum(xyz2 * xyz2, axis=1, keepdims=True)
    x1sq = jnp.sum(xyz1t * xyz1t, axis=0, keepdims=True)
    d = x2sq - 2.0 * cross + x1sq
    idx = jnp.argmin(d, axis=1)

    onehot = (jax.lax.broadcasted_iota(jnp.int32, (TILE, N1), 1)
              == idx[:, None]).astype(jnp.float32)
    nearest = jnp.dot(onehot, f1, preferred_element_type=jnp.float32)

    out_ref[0, :, 0:3] = xyz2
    out_ref[0, :, 3:3 + C1] = nearest
    out_ref[0, :, 3 + C1:] = f2


def kernel(x0, x1):
    xyz1t = jnp.transpose(x0[:, :, 0:3], (0, 2, 1))
    f1 = x0[:, :, 3:]
    out = pl.pallas_call(
        _fused_body,
        grid=(B, N2 // TILE),
        in_specs=[
            pl.BlockSpec((1, 3, N1), lambda b, t: (b, 0, 0)),
            pl.BlockSpec((1, N1, C1), lambda b, t: (b, 0, 0)),
            pl.BlockSpec((1, TILE, 3 + C2), lambda b, t: (b, t, 0)),
        ],
        out_specs=pl.BlockSpec((1, TILE, OUTC), lambda b, t: (b, t, 0)),
        out_shape=jax.ShapeDtypeStruct((B, N2, OUTC), jnp.float32),
    )(xyz1t, f1, x1)
    return (out, x1[:, :, 0:3])
